# baseline (speedup 1.0000x reference)
import functools

import jax
import jax.numpy as jnp
from jax import lax
from jax.experimental import pallas as pl
from jax.experimental.pallas import tpu as pltpu
from jax.experimental.pallas import tpu_sc as plsc

N = 10000
E = 320000
F_IN = 128
HID = 128
NCLS = 64

NC = 2
NS = 16
NW = NC * NS
EPW = E // NW
W = 128
EPWP = 10240
NWIN = EPWP // W
NPAD = 10240
RPS = NPAD // NS

DPAD = 10240
DRPS = DPAD // NS


def _mesh():
    return plsc.VectorSubcoreMesh(core_axis_name="c", subcore_axis_name="s")


@functools.partial(
    pl.kernel,
    out_type=jax.ShapeDtypeStruct((NC * DPAD,), jnp.float32),
    mesh=_mesh(),
    scratch_types=[
        pltpu.VMEM((NWIN, W), jnp.int32),
        pltpu.VMEM((W,), jnp.float32),
        pltpu.VMEM_SHARED((DPAD,), jnp.float32),
    ],
)
def _sc_degree(dst_hbm, z_hbm, ones_hbm, out_hbm, idx_v, ones_v, acc):
    c = lax.axis_index("c")
    s = lax.axis_index("s")
    wid = s * NC + c
    pltpu.sync_copy(ones_hbm, ones_v)
    pltpu.sync_copy(dst_hbm.at[wid], idx_v)
    pltpu.sync_copy(z_hbm.at[pl.ds(s * DRPS, DRPS)],
                    acc.at[pl.ds(s * DRPS, DRPS)])
    plsc.subcore_barrier()

    @pl.loop(0, NWIN)
    def _(w):
        pltpu.sync_copy(ones_v, acc.at[idx_v.at[w]], add=True)

    plsc.subcore_barrier()
    pltpu.sync_copy(acc.at[pl.ds(s * DRPS, DRPS)],
                    out_hbm.at[pl.ds(c * DPAD + s * DRPS, DRPS)])


NIB = 4
NRB = 2


def _make_sc_propagate(f):
    @functools.partial(
        pl.kernel,
        out_type=jax.ShapeDtypeStruct((NC * NPAD, f), jnp.float32),
        mesh=_mesh(),
        scratch_types=[
            pltpu.VMEM((NWIN, W), jnp.int32),
            [pltpu.VMEM((W,), jnp.int32)] * NIB,
            [pltpu.VMEM((W, f), jnp.float32)] * NRB,
            [pltpu.SemaphoreType.DMA] * NIB,
            [pltpu.SemaphoreType.DMA] * NRB,
            pltpu.VMEM_SHARED((NPAD, f), jnp.float32),
        ],
    )
    def _sc_propagate(y_hbm, src_hbm, dst_hbm, z_hbm, out_hbm,
                      dst_v, src_v, rows_v, isems, rsems, acc):
        c = lax.axis_index("c")
        s = lax.axis_index("s")
        wid = s * NC + c
        pltpu.sync_copy(dst_hbm.at[wid], dst_v)
        pltpu.sync_copy(z_hbm.at[pl.ds(s * RPS, RPS)],
                        acc.at[pl.ds(s * RPS, RPS)])
        plsc.subcore_barrier()

        for b in range(NIB):
            pltpu.async_copy(src_hbm.at[wid, b], src_v[b], isems[b])
        pltpu.make_async_copy(src_hbm.at[wid, 0], src_v[0], isems[0]).wait()
        pltpu.async_copy(y_hbm.at[src_v[0]], rows_v[0], rsems[0])

        @pl.loop(0, NWIN, step=NIB)
        def _(g):
            for b in range(NIB):
                w = g + b
                rb = b % NRB
                pltpu.make_async_copy(
                    y_hbm.at[src_v[b]], rows_v[rb], rsems[rb]).wait()

                @pl.when(w + NIB < NWIN)
                def _():
                    pltpu.async_copy(src_hbm.at[wid, w + NIB],
                                     src_v[b], isems[b])

                @pl.when(w + 1 < NWIN)
                def _():
                    b1 = (b + 1) % NIB
                    rb1 = (b + 1) % NRB
                    pltpu.make_async_copy(
                        src_hbm.at[wid, w + 1], src_v[b1], isems[b1]).wait()
                    pltpu.async_copy(y_hbm.at[src_v[b1]], rows_v[rb1],
                                     rsems[rb1])

                pltpu.sync_copy(rows_v[rb], acc.at[dst_v.at[w]], add=True)

        plsc.subcore_barrier()
        pltpu.sync_copy(acc.at[pl.ds(s * RPS, RPS)],
                        out_hbm.at[pl.ds(c * NPAD + s * RPS, RPS)])

    return _sc_propagate


_sc_prop_hid = _make_sc_propagate(HID)
_sc_prop_cls = _make_sc_propagate(HID)


def _dot(a, b):
    return lax.dot_general(a, b, (((1,), (0,)), ((), ())),
                           precision=lax.Precision.HIGHEST,
                           preferred_element_type=jnp.float32)


def _dinv(d0, d1):
    return lax.rsqrt(d0 + d1 + 1.0)


def _tc_xw_body(x_ref, w_ref, o_ref):
    o_ref[...] = _dot(x_ref[...], w_ref[...])


def _tc_scale_body(xw_ref, d0_ref, d1_ref, o_ref):
    o_ref[...] = xw_ref[...] * _dinv(d0_ref[...], d1_ref[...])


def _tc_mid_body(p_ref, y_ref, d0_ref, d1_ref, b_ref, w_ref, o_ref):
    dinv = _dinv(d0_ref[...], d1_ref[...])
    t = p_ref[0:N] + p_ref[NPAD:NPAD + N] + y_ref[...]
    h = jnp.maximum(dinv * t + b_ref[...], 0.0)
    o_ref[...] = dinv * _dot(h, w_ref[...])


def _tc_final_body(p_ref, y_ref, d0_ref, d1_ref, b_ref, ls_ref, z_ref, sm_ref):
    dinv = _dinv(d0_ref[...], d1_ref[...])
    t = p_ref[0:N, 0:NCLS] + p_ref[NPAD:NPAD + N, 0:NCLS] + y_ref[0:N, 0:NCLS]
    z = dinv * t + b_ref[...]
    z_ref[...] = z
    m1 = jnp.max(z, axis=1, keepdims=True)
    e1 = jnp.exp(z - m1)
    ls_ref[...] = z - m1 - jnp.log(jnp.sum(e1, axis=1, keepdims=True))
    m0 = jnp.max(z, axis=0, keepdims=True)
    e0 = jnp.exp(z - m0)
    sm_ref[...] = e0 / jnp.sum(e0, axis=0, keepdims=True)


def _f32(shape):
    return jax.ShapeDtypeStruct(shape, jnp.float32)


_tc_xw = pl.pallas_call(_tc_xw_body, out_shape=_f32((N, HID)))
_tc_scale = pl.pallas_call(_tc_scale_body, out_shape=_f32((N, HID)))
_tc_mid = pl.pallas_call(_tc_mid_body, out_shape=_f32((N, HID)))
_tc_final = pl.pallas_call(
    _tc_final_body,
    out_shape=(_f32((N, NCLS)), _f32((N, NCLS)), _f32((N, NCLS))),
)


def kernel(x, edge_index, W1, b1, W2, b2):
    src = edge_index[0]
    dst = edge_index[1]
    dzeros = jnp.zeros((DPAD,), jnp.float32)
    ones = jnp.ones((W,), jnp.float32)
    pz_hid = jnp.zeros((NPAD, HID), jnp.float32)
    pz_cls = jnp.zeros((NPAD, HID), jnp.float32)

    pad = ((0, 0), (0, EPWP - EPW))
    src3 = jnp.pad(src.reshape(NW, EPW), pad).reshape(NW, NWIN, W)
    dst3 = jnp.pad(dst.reshape(NW, EPW), pad,
                   constant_values=NPAD - 1).reshape(NW, NWIN, W)
    dp = _sc_degree(dst3, dzeros, ones)
    xw1 = _tc_xw(x, W1)
    d0 = dp[0:N].reshape(N, 1)
    d1 = dp[DPAD:DPAD + N].reshape(N, 1)
    y1 = _tc_scale(xw1, d0, d1)
    p1 = _sc_prop_hid(y1, src3, dst3, pz_hid)
    w2p = jnp.pad(W2, ((0, 0), (0, HID - NCLS)))
    y2 = _tc_mid(p1, y1, d0, d1, b1.reshape(1, HID), w2p)
    p2 = _sc_prop_cls(y2, src3, dst3, pz_cls)
    ls, z, sm = _tc_final(p2, y2, d0, d1, b2.reshape(1, NCLS))
    return (ls, z, sm)


# speedup vs baseline: 1.0007x; 1.0007x over previous
import functools

import jax
import jax.numpy as jnp
from jax import lax
from jax.experimental import pallas as pl
from jax.experimental.pallas import tpu as pltpu
from jax.experimental.pallas import tpu_sc as plsc

N = 10000
E = 320000
F_IN = 128
HID = 128
NCLS = 64

NC = 2
NS = 16
NW = NC * NS
EPW = E // NW
W = 128
EPWP = 10240
NWIN = EPWP // W
NPAD = 10240
RPS = NPAD // NS

DPAD = 10240
DRPS = DPAD // NS


def _mesh():
    return plsc.VectorSubcoreMesh(core_axis_name="c", subcore_axis_name="s")


@functools.partial(
    pl.kernel,
    out_type=jax.ShapeDtypeStruct((NC * DPAD,), jnp.float32),
    mesh=_mesh(),
    scratch_types=[
        pltpu.VMEM((NWIN, W), jnp.int32),
        pltpu.VMEM((W,), jnp.float32),
        pltpu.VMEM_SHARED((DPAD,), jnp.float32),
    ],
)
def _sc_degree(dst_hbm, z_hbm, ones_hbm, out_hbm, idx_v, ones_v, acc):
    c = lax.axis_index("c")
    s = lax.axis_index("s")
    wid = s * NC + c
    pltpu.sync_copy(ones_hbm, ones_v)
    pltpu.sync_copy(dst_hbm.at[wid], idx_v)
    pltpu.sync_copy(z_hbm.at[pl.ds(s * DRPS, DRPS)],
                    acc.at[pl.ds(s * DRPS, DRPS)])
    plsc.subcore_barrier()

    @pl.loop(0, NWIN)
    def _(w):
        pltpu.sync_copy(ones_v, acc.at[idx_v.at[w]], add=True)

    plsc.subcore_barrier()
    pltpu.sync_copy(acc.at[pl.ds(s * DRPS, DRPS)],
                    out_hbm.at[pl.ds(c * DPAD + s * DRPS, DRPS)])


NIB = 4
NRB = 2


def _make_sc_propagate(f):
    @functools.partial(
        pl.kernel,
        out_type=jax.ShapeDtypeStruct((NC * NPAD, f), jnp.float32),
        mesh=_mesh(),
        scratch_types=[
            pltpu.VMEM((NWIN, W), jnp.int32),
            [pltpu.VMEM((W,), jnp.int32)] * NIB,
            [pltpu.VMEM((W, f), jnp.float32)] * NRB,
            [pltpu.SemaphoreType.DMA] * NIB,
            [pltpu.SemaphoreType.DMA] * NRB,
            [pltpu.SemaphoreType.DMA] * NRB,
            pltpu.VMEM_SHARED((NPAD, f), jnp.float32),
        ],
    )
    def _sc_propagate(y_hbm, src_hbm, dst_hbm, z_hbm, out_hbm,
                      dst_v, src_v, rows_v, isems, rsems, ssems, acc):
        c = lax.axis_index("c")
        s = lax.axis_index("s")
        wid = s * NC + c
        pltpu.sync_copy(dst_hbm.at[wid], dst_v)
        pltpu.sync_copy(z_hbm.at[pl.ds(s * RPS, RPS)],
                        acc.at[pl.ds(s * RPS, RPS)])
        plsc.subcore_barrier()

        for b in range(NIB):
            pltpu.async_copy(src_hbm.at[wid, b], src_v[b], isems[b])
        pltpu.make_async_copy(src_hbm.at[wid, 0], src_v[0], isems[0]).wait()
        pltpu.async_copy(y_hbm.at[src_v[0]], rows_v[0], rsems[0])

        @pl.loop(0, NWIN, step=NIB)
        def _(g):
            for b in range(NIB):
                w = g + b
                rb = b % NRB
                b1 = (b + 1) % NIB
                rb1 = (b + 1) % NRB
                pltpu.make_async_copy(
                    y_hbm.at[src_v[b]], rows_v[rb], rsems[rb]).wait()

                @pl.when(w + NIB < NWIN)
                def _():
                    pltpu.async_copy(src_hbm.at[wid, w + NIB],
                                     src_v[b], isems[b])

                @pl.when(w > 0)
                def _():
                    pltpu.make_async_copy(
                        rows_v[rb1], acc.at[dst_v.at[w - 1]],
                        ssems[rb1]).wait()

                @pl.when(w + 1 < NWIN)
                def _():
                    pltpu.make_async_copy(
                        src_hbm.at[wid, w + 1], src_v[b1], isems[b1]).wait()
                    pltpu.async_copy(y_hbm.at[src_v[b1]], rows_v[rb1],
                                     rsems[rb1])

                pltpu.async_copy(rows_v[rb], acc.at[dst_v.at[w]],
                                 ssems[rb], add=True)

        pltpu.make_async_copy(
            rows_v[(NWIN - 1) % NRB], acc.at[dst_v.at[NWIN - 1]],
            ssems[(NWIN - 1) % NRB]).wait()
        plsc.subcore_barrier()
        pltpu.sync_copy(acc.at[pl.ds(s * RPS, RPS)],
                        out_hbm.at[pl.ds(c * NPAD + s * RPS, RPS)])

    return _sc_propagate


_sc_prop_hid = _make_sc_propagate(HID)
_sc_prop_cls = _make_sc_propagate(HID)


def _dot(a, b):
    return lax.dot_general(a, b, (((1,), (0,)), ((), ())),
                           precision=lax.Precision.HIGHEST,
                           preferred_element_type=jnp.float32)


def _dinv(d0, d1):
    return lax.rsqrt(d0 + d1 + 1.0)


def _tc_xw_body(x_ref, w_ref, o_ref):
    o_ref[...] = _dot(x_ref[...], w_ref[...])


def _tc_scale_body(xw_ref, d0_ref, d1_ref, o_ref):
    o_ref[...] = xw_ref[...] * _dinv(d0_ref[...], d1_ref[...])


def _tc_mid_body(p_ref, y_ref, d0_ref, d1_ref, b_ref, w_ref, o_ref):
    dinv = _dinv(d0_ref[...], d1_ref[...])
    t = p_ref[0:N] + p_ref[NPAD:NPAD + N] + y_ref[...]
    h = jnp.maximum(dinv * t + b_ref[...], 0.0)
    o_ref[...] = dinv * _dot(h, w_ref[...])


def _tc_final_body(p_ref, y_ref, d0_ref, d1_ref, b_ref, ls_ref, z_ref, sm_ref):
    dinv = _dinv(d0_ref[...], d1_ref[...])
    t = p_ref[0:N, 0:NCLS] + p_ref[NPAD:NPAD + N, 0:NCLS] + y_ref[0:N, 0:NCLS]
    z = dinv * t + b_ref[...]
    z_ref[...] = z
    m1 = jnp.max(z, axis=1, keepdims=True)
    e1 = jnp.exp(z - m1)
    ls_ref[...] = z - m1 - jnp.log(jnp.sum(e1, axis=1, keepdims=True))
    m0 = jnp.max(z, axis=0, keepdims=True)
    e0 = jnp.exp(z - m0)
    sm_ref[...] = e0 / jnp.sum(e0, axis=0, keepdims=True)


def _f32(shape):
    return jax.ShapeDtypeStruct(shape, jnp.float32)


_tc_xw = pl.pallas_call(_tc_xw_body, out_shape=_f32((N, HID)))
_tc_scale = pl.pallas_call(_tc_scale_body, out_shape=_f32((N, HID)))
_tc_mid = pl.pallas_call(_tc_mid_body, out_shape=_f32((N, HID)))
_tc_final = pl.pallas_call(
    _tc_final_body,
    out_shape=(_f32((N, NCLS)), _f32((N, NCLS)), _f32((N, NCLS))),
)


def kernel(x, edge_index, W1, b1, W2, b2):
    src = edge_index[0]
    dst = edge_index[1]
    dzeros = jnp.zeros((DPAD,), jnp.float32)
    ones = jnp.ones((W,), jnp.float32)
    pz_hid = jnp.zeros((NPAD, HID), jnp.float32)
    pz_cls = jnp.zeros((NPAD, HID), jnp.float32)

    pad = ((0, 0), (0, EPWP - EPW))
    src3 = jnp.pad(src.reshape(NW, EPW), pad).reshape(NW, NWIN, W)
    dst3 = jnp.pad(dst.reshape(NW, EPW), pad,
                   constant_values=NPAD - 1).reshape(NW, NWIN, W)
    dp = _sc_degree(dst3, dzeros, ones)
    xw1 = _tc_xw(x, W1)
    d0 = dp[0:N].reshape(N, 1)
    d1 = dp[DPAD:DPAD + N].reshape(N, 1)
    y1 = _tc_scale(xw1, d0, d1)
    p1 = _sc_prop_hid(y1, src3, dst3, pz_hid)
    w2p = jnp.pad(W2, ((0, 0), (0, HID - NCLS)))
    y2 = _tc_mid(p1, y1, d0, d1, b1.reshape(1, HID), w2p)
    p2 = _sc_prop_cls(y2, src3, dst3, pz_cls)
    ls, z, sm = _tc_final(p2, y2, d0, d1, b2.reshape(1, NCLS))
    return (ls, z, sm)


# speedup vs baseline: 1.0476x; 1.0468x over previous
import functools

import jax
import jax.numpy as jnp
from jax import lax
from jax.experimental import pallas as pl
from jax.experimental.pallas import tpu as pltpu
from jax.experimental.pallas import tpu_sc as plsc

N = 10000
E = 320000
F_IN = 128
HID = 128
NCLS = 64

NC = 2
NS = 16
NW = NC * NS
EPW = E // NW
W = 128
EPWP = 10240
NWIN = EPWP // W
NPAD = 10240
RPS = NPAD // NS

DPAD = 10240
DRPS = DPAD // NS


def _mesh():
    return plsc.VectorSubcoreMesh(core_axis_name="c", subcore_axis_name="s")


@functools.partial(
    pl.kernel,
    out_type=jax.ShapeDtypeStruct((NC * DPAD,), jnp.float32),
    mesh=_mesh(),
    scratch_types=[
        pltpu.VMEM((NWIN, W), jnp.int32),
        pltpu.VMEM((W,), jnp.float32),
        pltpu.VMEM_SHARED((DPAD,), jnp.float32),
    ],
)
def _sc_degree(dst_hbm, z_hbm, ones_hbm, out_hbm, idx_v, ones_v, acc):
    c = lax.axis_index("c")
    s = lax.axis_index("s")
    wid = s * NC + c
    pltpu.sync_copy(ones_hbm, ones_v)
    pltpu.sync_copy(dst_hbm.at[wid], idx_v)
    pltpu.sync_copy(z_hbm.at[pl.ds(s * DRPS, DRPS)],
                    acc.at[pl.ds(s * DRPS, DRPS)])
    plsc.subcore_barrier()

    @pl.loop(0, NWIN)
    def _(w):
        pltpu.sync_copy(ones_v, acc.at[idx_v.at[w]], add=True)

    plsc.subcore_barrier()
    pltpu.sync_copy(acc.at[pl.ds(s * DRPS, DRPS)],
                    out_hbm.at[pl.ds(c * DPAD + s * DRPS, DRPS)])


NIB = 4
NRB = 2


def _make_sc_propagate(f):
    @functools.partial(
        pl.kernel,
        out_type=jax.ShapeDtypeStruct((NC * NPAD, f), jnp.float32),
        mesh=_mesh(),
        scratch_types=[
            pltpu.VMEM((NWIN, W), jnp.int32),
            [pltpu.VMEM((W,), jnp.int32)] * NIB,
            [pltpu.VMEM((W, f), jnp.float32)] * NRB,
            [pltpu.SemaphoreType.DMA] * NIB,
            [pltpu.SemaphoreType.DMA] * NRB,
            [pltpu.SemaphoreType.DMA] * NRB,
            pltpu.VMEM_SHARED((NPAD, f), jnp.float32),
        ],
    )
    def _sc_propagate(y_hbm, src_hbm, dst_hbm, z_hbm, out_hbm,
                      dst_v, src_v, rows_v, isems, rsems, ssems, acc):
        c = lax.axis_index("c")
        s = lax.axis_index("s")
        wid = s * NC + c
        pltpu.sync_copy(dst_hbm.at[wid], dst_v)
        pltpu.sync_copy(z_hbm.at[pl.ds(s * RPS, RPS)],
                        acc.at[pl.ds(s * RPS, RPS)])
        plsc.subcore_barrier()

        for b in range(NIB):
            pltpu.async_copy(src_hbm.at[wid, b], src_v[b], isems[b])
        for b in range(NRB):
            pltpu.make_async_copy(src_hbm.at[wid, b], src_v[b],
                                  isems[b]).wait()
            pltpu.async_copy(y_hbm.at[src_v[b]], rows_v[b], rsems[b])

        @pl.loop(0, NWIN, step=NIB)
        def _(g):
            for b in range(NIB):
                w = g + b
                rb = b % NRB
                b2 = (b + NRB) % NIB
                pltpu.make_async_copy(
                    y_hbm.at[src_v[b]], rows_v[rb], rsems[rb]).wait()
                pltpu.sync_copy(rows_v[rb], acc.at[dst_v.at[w]], add=True)

                @pl.when(w + NIB < NWIN)
                def _():
                    pltpu.async_copy(src_hbm.at[wid, w + NIB],
                                     src_v[b], isems[b])

                @pl.when(w + NRB < NWIN)
                def _():
                    pltpu.make_async_copy(
                        src_hbm.at[wid, w + NRB], src_v[b2], isems[b2]).wait()
                    pltpu.async_copy(y_hbm.at[src_v[b2]], rows_v[rb],
                                     rsems[rb])

        plsc.subcore_barrier()
        pltpu.sync_copy(acc.at[pl.ds(s * RPS, RPS)],
                        out_hbm.at[pl.ds(c * NPAD + s * RPS, RPS)])

    return _sc_propagate


_sc_prop_hid = _make_sc_propagate(HID)
_sc_prop_cls = _make_sc_propagate(HID)


def _dot(a, b):
    return lax.dot_general(a, b, (((1,), (0,)), ((), ())),
                           precision=lax.Precision.HIGHEST,
                           preferred_element_type=jnp.float32)


def _dinv(d0, d1):
    return lax.rsqrt(d0 + d1 + 1.0)


def _tc_xw_body(x_ref, w_ref, o_ref):
    o_ref[...] = _dot(x_ref[...], w_ref[...])


def _tc_scale_body(xw_ref, d0_ref, d1_ref, o_ref):
    o_ref[...] = xw_ref[...] * _dinv(d0_ref[...], d1_ref[...])


def _tc_mid_body(p_ref, y_ref, d0_ref, d1_ref, b_ref, w_ref, o_ref):
    dinv = _dinv(d0_ref[...], d1_ref[...])
    t = p_ref[0:N] + p_ref[NPAD:NPAD + N] + y_ref[...]
    h = jnp.maximum(dinv * t + b_ref[...], 0.0)
    o_ref[...] = dinv * _dot(h, w_ref[...])


def _tc_final_body(p_ref, y_ref, d0_ref, d1_ref, b_ref, ls_ref, z_ref, sm_ref):
    dinv = _dinv(d0_ref[...], d1_ref[...])
    t = p_ref[0:N, 0:NCLS] + p_ref[NPAD:NPAD + N, 0:NCLS] + y_ref[0:N, 0:NCLS]
    z = dinv * t + b_ref[...]
    z_ref[...] = z
    m1 = jnp.max(z, axis=1, keepdims=True)
    e1 = jnp.exp(z - m1)
    ls_ref[...] = z - m1 - jnp.log(jnp.sum(e1, axis=1, keepdims=True))
    m0 = jnp.max(z, axis=0, keepdims=True)
    e0 = jnp.exp(z - m0)
    sm_ref[...] = e0 / jnp.sum(e0, axis=0, keepdims=True)


def _f32(shape):
    return jax.ShapeDtypeStruct(shape, jnp.float32)


_tc_xw = pl.pallas_call(_tc_xw_body, out_shape=_f32((N, HID)))
_tc_scale = pl.pallas_call(_tc_scale_body, out_shape=_f32((N, HID)))
_tc_mid = pl.pallas_call(_tc_mid_body, out_shape=_f32((N, HID)))
_tc_final = pl.pallas_call(
    _tc_final_body,
    out_shape=(_f32((N, NCLS)), _f32((N, NCLS)), _f32((N, NCLS))),
)


def kernel(x, edge_index, W1, b1, W2, b2):
    src = edge_index[0]
    dst = edge_index[1]
    dzeros = jnp.zeros((DPAD,), jnp.float32)
    ones = jnp.ones((W,), jnp.float32)
    pz_hid = jnp.zeros((NPAD, HID), jnp.float32)
    pz_cls = jnp.zeros((NPAD, HID), jnp.float32)

    pad = ((0, 0), (0, EPWP - EPW))
    src3 = jnp.pad(src.reshape(NW, EPW), pad).reshape(NW, NWIN, W)
    dst3 = jnp.pad(dst.reshape(NW, EPW), pad,
                   constant_values=NPAD - 1).reshape(NW, NWIN, W)
    dp = _sc_degree(dst3, dzeros, ones)
    xw1 = _tc_xw(x, W1)
    d0 = dp[0:N].reshape(N, 1)
    d1 = dp[DPAD:DPAD + N].reshape(N, 1)
    y1 = _tc_scale(xw1, d0, d1)
    p1 = _sc_prop_hid(y1, src3, dst3, pz_hid)
    w2p = jnp.pad(W2, ((0, 0), (0, HID - NCLS)))
    y2 = _tc_mid(p1, y1, d0, d1, b1.reshape(1, HID), w2p)
    p2 = _sc_prop_cls(y2, src3, dst3, pz_cls)
    ls, z, sm = _tc_final(p2, y2, d0, d1, b2.reshape(1, NCLS))
    return (ls, z, sm)


# speedup vs baseline: 2.6470x; 2.5268x over previous
import functools

import jax
import jax.numpy as jnp
from jax import lax
from jax.experimental import pallas as pl
from jax.experimental.pallas import tpu as pltpu
from jax.experimental.pallas import tpu_sc as plsc

N = 10000
E = 320000
F_IN = 128
HID = 128
NCLS = 64

NC = 2
NS = 16
NW = NC * NS
EPW = E // NW
W = 80
EPWP = 10000
NWIN = EPWP // W
NPAD = 10240
RPS = NPAD // NS

DPAD = 10240
DRPS = DPAD // NS


def _mesh():
    return plsc.VectorSubcoreMesh(core_axis_name="c", subcore_axis_name="s")


@functools.partial(
    pl.kernel,
    out_type=jax.ShapeDtypeStruct((NC * DPAD,), jnp.float32),
    mesh=_mesh(),
    scratch_types=[
        pltpu.VMEM((NWIN, W), jnp.int32),
        pltpu.VMEM((W,), jnp.float32),
        pltpu.VMEM_SHARED((DPAD,), jnp.float32),
    ],
)
def _sc_degree(dst_hbm, z_hbm, ones_hbm, out_hbm, idx_v, ones_v, acc):
    c = lax.axis_index("c")
    s = lax.axis_index("s")
    wid = s * NC + c
    pltpu.sync_copy(ones_hbm, ones_v)
    pltpu.sync_copy(dst_hbm.at[wid], idx_v)
    pltpu.sync_copy(z_hbm.at[pl.ds(s * DRPS, DRPS)],
                    acc.at[pl.ds(s * DRPS, DRPS)])
    plsc.subcore_barrier()

    @pl.loop(0, NWIN)
    def _(w):
        pltpu.sync_copy(ones_v, acc.at[idx_v.at[w]], add=True)

    plsc.subcore_barrier()
    pltpu.sync_copy(acc.at[pl.ds(s * DRPS, DRPS)],
                    out_hbm.at[pl.ds(c * DPAD + s * DRPS, DRPS)])


NIB = 4
NRB = 2


def _make_sc_propagate(f):
    @functools.partial(
        pl.kernel,
        out_type=jax.ShapeDtypeStruct((NC * NPAD, f), jnp.float32),
        mesh=_mesh(),
        scratch_types=[
            pltpu.VMEM((NWIN, W), jnp.int32),
            [pltpu.VMEM((W,), jnp.int32)] * NIB,
            [pltpu.VMEM((W, f), jnp.float32)] * NRB,
            [pltpu.SemaphoreType.DMA] * NIB,
            [pltpu.SemaphoreType.DMA] * NRB,
            [pltpu.SemaphoreType.DMA] * NRB,
            pltpu.VMEM_SHARED((NPAD, f), jnp.float32),
        ],
    )
    def _sc_propagate(y_hbm, src_hbm, dst_hbm, z_hbm, out_hbm,
                      dst_v, src_v, rows_v, isems, rsems, ssems, acc):
        c = lax.axis_index("c")
        s = lax.axis_index("s")
        wid = s * NC + c
        pltpu.sync_copy(dst_hbm.at[wid], dst_v)
        pltpu.sync_copy(z_hbm.at[pl.ds(s * RPS, RPS)],
                        acc.at[pl.ds(s * RPS, RPS)])
        plsc.subcore_barrier()

        for b in range(NIB):
            pltpu.async_copy(src_hbm.at[wid, b], src_v[b], isems[b])
        for b in range(NRB):
            pltpu.make_async_copy(src_hbm.at[wid, b], src_v[b],
                                  isems[b]).wait()
            pltpu.async_copy(y_hbm.at[src_v[b]], rows_v[b], rsems[b])

        @pl.loop(0, NWIN, step=NIB)
        def _(g):
            for b in range(NIB):
                w = g + b
                rb = b % NRB
                b2 = (b + NRB) % NIB
                @pl.when(w < NWIN)
                def _():
                    pltpu.make_async_copy(
                        y_hbm.at[src_v[b]], rows_v[rb], rsems[rb]).wait()
                    pltpu.sync_copy(rows_v[rb], acc.at[dst_v.at[w]], add=True)

                @pl.when(w + NIB < NWIN)
                def _():
                    pltpu.async_copy(src_hbm.at[wid, w + NIB],
                                     src_v[b], isems[b])

                @pl.when(w + NRB < NWIN)
                def _():
                    pltpu.make_async_copy(
                        src_hbm.at[wid, w + NRB], src_v[b2], isems[b2]).wait()
                    pltpu.async_copy(y_hbm.at[src_v[b2]], rows_v[rb],
                                     rsems[rb])

        plsc.subcore_barrier()
        pltpu.sync_copy(acc.at[pl.ds(s * RPS, RPS)],
                        out_hbm.at[pl.ds(c * NPAD + s * RPS, RPS)])

    return _sc_propagate


_sc_prop_hid = _make_sc_propagate(HID)
_sc_prop_cls = _make_sc_propagate(HID)


def _dot(a, b):
    return lax.dot_general(a, b, (((1,), (0,)), ((), ())),
                           precision=lax.Precision.HIGHEST,
                           preferred_element_type=jnp.float32)


def _dinv(d0, d1):
    return lax.rsqrt(d0 + d1 + 1.0)


def _tc_xw_body(x_ref, w_ref, o_ref):
    o_ref[...] = _dot(x_ref[...], w_ref[...])


def _tc_scale_body(xw_ref, d0_ref, d1_ref, o_ref):
    o_ref[...] = xw_ref[...] * _dinv(d0_ref[...], d1_ref[...])


def _tc_mid_body(p_ref, y_ref, d0_ref, d1_ref, b_ref, w_ref, o_ref):
    dinv = _dinv(d0_ref[...], d1_ref[...])
    t = p_ref[0:N] + p_ref[NPAD:NPAD + N] + y_ref[...]
    h = jnp.maximum(dinv * t + b_ref[...], 0.0)
    o_ref[...] = dinv * _dot(h, w_ref[...])


def _tc_final_body(p_ref, y_ref, d0_ref, d1_ref, b_ref, ls_ref, z_ref, sm_ref):
    dinv = _dinv(d0_ref[...], d1_ref[...])
    t = p_ref[0:N, 0:NCLS] + p_ref[NPAD:NPAD + N, 0:NCLS] + y_ref[0:N, 0:NCLS]
    z = dinv * t + b_ref[...]
    z_ref[...] = z
    m1 = jnp.max(z, axis=1, keepdims=True)
    e1 = jnp.exp(z - m1)
    ls_ref[...] = z - m1 - jnp.log(jnp.sum(e1, axis=1, keepdims=True))
    m0 = jnp.max(z, axis=0, keepdims=True)
    e0 = jnp.exp(z - m0)
    sm_ref[...] = e0 / jnp.sum(e0, axis=0, keepdims=True)


def _f32(shape):
    return jax.ShapeDtypeStruct(shape, jnp.float32)


_tc_xw = pl.pallas_call(_tc_xw_body, out_shape=_f32((N, HID)))
_tc_scale = pl.pallas_call(_tc_scale_body, out_shape=_f32((N, HID)))
_tc_mid = pl.pallas_call(_tc_mid_body, out_shape=_f32((N, HID)))
_tc_final = pl.pallas_call(
    _tc_final_body,
    out_shape=(_f32((N, NCLS)), _f32((N, NCLS)), _f32((N, NCLS))),
)


def kernel(x, edge_index, W1, b1, W2, b2):
    src = edge_index[0]
    dst = edge_index[1]
    dzeros = jnp.zeros((DPAD,), jnp.float32)
    ones = jnp.ones((W,), jnp.float32)
    pz_hid = jnp.zeros((NPAD, HID), jnp.float32)
    pz_cls = jnp.zeros((NPAD, HID), jnp.float32)

    pad = ((0, 0), (0, EPWP - EPW))
    src3 = jnp.pad(src.reshape(NW, EPW), pad).reshape(NW, NWIN, W)
    dst3 = jnp.pad(dst.reshape(NW, EPW), pad,
                   constant_values=NPAD - 1).reshape(NW, NWIN, W)
    dp = _sc_degree(dst3, dzeros, ones)
    xw1 = _tc_xw(x, W1)
    d0 = dp[0:N].reshape(N, 1)
    d1 = dp[DPAD:DPAD + N].reshape(N, 1)
    y1 = _tc_scale(xw1, d0, d1)
    p1 = _sc_prop_hid(y1, src3, dst3, pz_hid)
    w2p = jnp.pad(W2, ((0, 0), (0, HID - NCLS)))
    y2 = _tc_mid(p1, y1, d0, d1, b1.reshape(1, HID), w2p)
    p2 = _sc_prop_cls(y2, src3, dst3, pz_cls)
    ls, z, sm = _tc_final(p2, y2, d0, d1, b2.reshape(1, NCLS))
    return (ls, z, sm)


# speedup vs baseline: 3.0159x; 1.1393x over previous
import functools

import jax
import jax.numpy as jnp
from jax import lax
from jax.experimental import pallas as pl
from jax.experimental.pallas import tpu as pltpu
from jax.experimental.pallas import tpu_sc as plsc

N = 10000
E = 320000
F_IN = 128
HID = 128
NCLS = 64

NC = 2
NS = 16
NW = NC * NS
EPW = E // NW
W = 80
EPWP = 10000
NWIN = EPWP // W
NPAD = 10240
RPS = NPAD // NS

DPAD = 10240
DRPS = DPAD // NS


def _mesh():
    return plsc.VectorSubcoreMesh(core_axis_name="c", subcore_axis_name="s")


@functools.partial(
    pl.kernel,
    out_type=jax.ShapeDtypeStruct((NC * DPAD,), jnp.float32),
    mesh=_mesh(),
    scratch_types=[
        pltpu.VMEM((NWIN, W), jnp.int32),
        pltpu.VMEM((W,), jnp.float32),
        pltpu.VMEM_SHARED((DPAD,), jnp.float32),
    ],
)
def _sc_degree(dst_hbm, z_hbm, ones_hbm, out_hbm, idx_v, ones_v, acc):
    c = lax.axis_index("c")
    s = lax.axis_index("s")
    wid = s * NC + c
    pltpu.sync_copy(ones_hbm, ones_v)
    pltpu.sync_copy(dst_hbm.at[wid], idx_v)
    pltpu.sync_copy(z_hbm.at[pl.ds(s * DRPS, DRPS)],
                    acc.at[pl.ds(s * DRPS, DRPS)])
    plsc.subcore_barrier()

    @pl.loop(0, NWIN)
    def _(w):
        pltpu.sync_copy(ones_v, acc.at[idx_v.at[w]], add=True)

    plsc.subcore_barrier()
    pltpu.sync_copy(acc.at[pl.ds(s * DRPS, DRPS)],
                    out_hbm.at[pl.ds(c * DPAD + s * DRPS, DRPS)])


NIB = 6
NRB = 3


def _make_sc_propagate(f):
    @functools.partial(
        pl.kernel,
        out_type=jax.ShapeDtypeStruct((NC * NPAD, f), jnp.float32),
        mesh=_mesh(),
        scratch_types=[
            pltpu.VMEM((NWIN, W), jnp.int32),
            [pltpu.VMEM((W,), jnp.int32)] * NIB,
            [pltpu.VMEM((W, f), jnp.float32)] * NRB,
            [pltpu.SemaphoreType.DMA] * NIB,
            [pltpu.SemaphoreType.DMA] * NRB,
            [pltpu.SemaphoreType.DMA] * NRB,
            pltpu.VMEM_SHARED((NPAD, f), jnp.float32),
        ],
    )
    def _sc_propagate(y_hbm, src_hbm, dst_hbm, z_hbm, out_hbm,
                      dst_v, src_v, rows_v, isems, rsems, ssems, acc):
        c = lax.axis_index("c")
        s = lax.axis_index("s")
        wid = s * NC + c
        pltpu.sync_copy(dst_hbm.at[wid], dst_v)
        pltpu.sync_copy(z_hbm.at[pl.ds(s * RPS, RPS)],
                        acc.at[pl.ds(s * RPS, RPS)])
        plsc.subcore_barrier()

        for b in range(NIB):
            pltpu.async_copy(src_hbm.at[wid, b], src_v[b], isems[b])
        for b in range(NRB):
            pltpu.make_async_copy(src_hbm.at[wid, b], src_v[b],
                                  isems[b]).wait()
            pltpu.async_copy(y_hbm.at[src_v[b]], rows_v[b], rsems[b])

        @pl.loop(0, NWIN, step=NIB)
        def _(g):
            for b in range(NIB):
                w = g + b
                rb = b % NRB
                b2 = (b + NRB) % NIB
                @pl.when(w < NWIN)
                def _():
                    pltpu.make_async_copy(
                        y_hbm.at[src_v[b]], rows_v[rb], rsems[rb]).wait()
                    pltpu.sync_copy(rows_v[rb], acc.at[dst_v.at[w]], add=True)

                @pl.when(w + NIB < NWIN)
                def _():
                    pltpu.async_copy(src_hbm.at[wid, w + NIB],
                                     src_v[b], isems[b])

                @pl.when(w + NRB < NWIN)
                def _():
                    pltpu.make_async_copy(
                        src_hbm.at[wid, w + NRB], src_v[b2], isems[b2]).wait()
                    pltpu.async_copy(y_hbm.at[src_v[b2]], rows_v[rb],
                                     rsems[rb])

        plsc.subcore_barrier()
        pltpu.sync_copy(acc.at[pl.ds(s * RPS, RPS)],
                        out_hbm.at[pl.ds(c * NPAD + s * RPS, RPS)])

    return _sc_propagate


_sc_prop_hid = _make_sc_propagate(HID)
_sc_prop_cls = _make_sc_propagate(HID)


def _dot(a, b):
    return lax.dot_general(a, b, (((1,), (0,)), ((), ())),
                           precision=lax.Precision.HIGHEST,
                           preferred_element_type=jnp.float32)


def _dinv(d0, d1):
    return lax.rsqrt(d0 + d1 + 1.0)


def _tc_xw_body(x_ref, w_ref, o_ref):
    o_ref[...] = _dot(x_ref[...], w_ref[...])


def _tc_scale_body(xw_ref, d0_ref, d1_ref, o_ref):
    o_ref[...] = xw_ref[...] * _dinv(d0_ref[...], d1_ref[...])


def _tc_mid_body(p_ref, y_ref, d0_ref, d1_ref, b_ref, w_ref, o_ref):
    dinv = _dinv(d0_ref[...], d1_ref[...])
    t = p_ref[0:N] + p_ref[NPAD:NPAD + N] + y_ref[...]
    h = jnp.maximum(dinv * t + b_ref[...], 0.0)
    o_ref[...] = dinv * _dot(h, w_ref[...])


def _tc_final_body(p_ref, y_ref, d0_ref, d1_ref, b_ref, ls_ref, z_ref, sm_ref):
    dinv = _dinv(d0_ref[...], d1_ref[...])
    t = p_ref[0:N, 0:NCLS] + p_ref[NPAD:NPAD + N, 0:NCLS] + y_ref[0:N, 0:NCLS]
    z = dinv * t + b_ref[...]
    z_ref[...] = z
    m1 = jnp.max(z, axis=1, keepdims=True)
    e1 = jnp.exp(z - m1)
    ls_ref[...] = z - m1 - jnp.log(jnp.sum(e1, axis=1, keepdims=True))
    m0 = jnp.max(z, axis=0, keepdims=True)
    e0 = jnp.exp(z - m0)
    sm_ref[...] = e0 / jnp.sum(e0, axis=0, keepdims=True)


def _f32(shape):
    return jax.ShapeDtypeStruct(shape, jnp.float32)


_tc_xw = pl.pallas_call(_tc_xw_body, out_shape=_f32((N, HID)))
_tc_scale = pl.pallas_call(_tc_scale_body, out_shape=_f32((N, HID)))
_tc_mid = pl.pallas_call(_tc_mid_body, out_shape=_f32((N, HID)))
_tc_final = pl.pallas_call(
    _tc_final_body,
    out_shape=(_f32((N, NCLS)), _f32((N, NCLS)), _f32((N, NCLS))),
)


def kernel(x, edge_index, W1, b1, W2, b2):
    src = edge_index[0]
    dst = edge_index[1]
    dzeros = jnp.zeros((DPAD,), jnp.float32)
    ones = jnp.ones((W,), jnp.float32)
    pz_hid = jnp.zeros((NPAD, HID), jnp.float32)
    pz_cls = jnp.zeros((NPAD, HID), jnp.float32)

    pad = ((0, 0), (0, EPWP - EPW))
    src3 = jnp.pad(src.reshape(NW, EPW), pad).reshape(NW, NWIN, W)
    dst3 = jnp.pad(dst.reshape(NW, EPW), pad,
                   constant_values=NPAD - 1).reshape(NW, NWIN, W)
    dp = _sc_degree(dst3, dzeros, ones)
    xw1 = _tc_xw(x, W1)
    d0 = dp[0:N].reshape(N, 1)
    d1 = dp[DPAD:DPAD + N].reshape(N, 1)
    y1 = _tc_scale(xw1, d0, d1)
    p1 = _sc_prop_hid(y1, src3, dst3, pz_hid)
    w2p = jnp.pad(W2, ((0, 0), (0, HID - NCLS)))
    y2 = _tc_mid(p1, y1, d0, d1, b1.reshape(1, HID), w2p)
    p2 = _sc_prop_cls(y2, src3, dst3, pz_cls)
    ls, z, sm = _tc_final(p2, y2, d0, d1, b2.reshape(1, NCLS))
    return (ls, z, sm)


# speedup vs baseline: 3.1065x; 1.0300x over previous
import functools

import jax
import jax.numpy as jnp
from jax import lax
from jax.experimental import pallas as pl
from jax.experimental.pallas import tpu as pltpu
from jax.experimental.pallas import tpu_sc as plsc

N = 10000
E = 320000
F_IN = 128
HID = 128
NCLS = 64

NC = 2
NS = 16
NW = NC * NS
EPW = E // NW
W = 80
EPWP = 10000
NWIN = EPWP // W
NPAD = 10240
RPS = NPAD // NS

DPAD = 10240
DRPS = DPAD // NS


def _mesh():
    return plsc.VectorSubcoreMesh(core_axis_name="c", subcore_axis_name="s")


@functools.partial(
    pl.kernel,
    out_type=jax.ShapeDtypeStruct((NC * DPAD,), jnp.float32),
    mesh=_mesh(),
    scratch_types=[
        pltpu.VMEM((NWIN, W), jnp.int32),
        pltpu.VMEM((W,), jnp.float32),
        pltpu.VMEM_SHARED((DPAD,), jnp.float32),
    ],
)
def _sc_degree(dst_hbm, z_hbm, ones_hbm, out_hbm, idx_v, ones_v, acc):
    c = lax.axis_index("c")
    s = lax.axis_index("s")
    wid = s * NC + c
    pltpu.sync_copy(ones_hbm, ones_v)
    pltpu.sync_copy(dst_hbm.at[wid], idx_v)
    pltpu.sync_copy(z_hbm.at[pl.ds(s * DRPS, DRPS)],
                    acc.at[pl.ds(s * DRPS, DRPS)])
    plsc.subcore_barrier()

    @pl.loop(0, NWIN)
    def _(w):
        pltpu.sync_copy(ones_v, acc.at[idx_v.at[w]], add=True)

    plsc.subcore_barrier()
    pltpu.sync_copy(acc.at[pl.ds(s * DRPS, DRPS)],
                    out_hbm.at[pl.ds(c * DPAD + s * DRPS, DRPS)])


NIB = 8
NRB = 4


def _make_sc_propagate(f):
    @functools.partial(
        pl.kernel,
        out_type=jax.ShapeDtypeStruct((NC * NPAD, f), jnp.float32),
        mesh=_mesh(),
        scratch_types=[
            [pltpu.VMEM((W,), jnp.int32)] * NIB,
            [pltpu.VMEM((W,), jnp.int32)] * NIB,
            [pltpu.VMEM((W, f), jnp.float32)] * NRB,
            [pltpu.SemaphoreType.DMA] * NIB,
            [pltpu.SemaphoreType.DMA] * NRB,
            pltpu.VMEM_SHARED((NPAD, f), jnp.float32),
        ],
    )
    def _sc_propagate(y_hbm, src_hbm, dst_hbm, z_hbm, out_hbm,
                      src_v, dst_v, rows_v, isems, rsems, acc):
        c = lax.axis_index("c")
        s = lax.axis_index("s")
        wid = s * NC + c
        pltpu.sync_copy(z_hbm.at[pl.ds(s * RPS, RPS)],
                        acc.at[pl.ds(s * RPS, RPS)])
        plsc.subcore_barrier()

        for b in range(NIB):
            pltpu.async_copy(src_hbm.at[wid, b], src_v[b], isems[b])
            pltpu.async_copy(dst_hbm.at[wid, b], dst_v[b], isems[b])
        for b in range(NRB):
            pltpu.make_async_copy(src_hbm.at[wid, b], src_v[b],
                                  isems[b]).wait()
            pltpu.make_async_copy(dst_hbm.at[wid, b], dst_v[b],
                                  isems[b]).wait()
            pltpu.async_copy(y_hbm.at[src_v[b]], rows_v[b], rsems[b])

        @pl.loop(0, NWIN, step=NIB)
        def _(g):
            for b in range(NIB):
                w = g + b
                rb = b % NRB
                b2 = (b + NRB) % NIB
                @pl.when(w < NWIN)
                def _():
                    pltpu.make_async_copy(
                        y_hbm.at[src_v[b]], rows_v[rb], rsems[rb]).wait()
                    pltpu.sync_copy(rows_v[rb], acc.at[dst_v[b]], add=True)

                @pl.when(w + NIB < NWIN)
                def _():
                    pltpu.async_copy(src_hbm.at[wid, w + NIB],
                                     src_v[b], isems[b])
                    pltpu.async_copy(dst_hbm.at[wid, w + NIB],
                                     dst_v[b], isems[b])

                @pl.when(w + NRB < NWIN)
                def _():
                    pltpu.make_async_copy(
                        src_hbm.at[wid, w + NRB], src_v[b2], isems[b2]).wait()
                    pltpu.make_async_copy(
                        dst_hbm.at[wid, w + NRB], dst_v[b2], isems[b2]).wait()
                    pltpu.async_copy(y_hbm.at[src_v[b2]], rows_v[rb],
                                     rsems[rb])

        plsc.subcore_barrier()
        pltpu.sync_copy(acc.at[pl.ds(s * RPS, RPS)],
                        out_hbm.at[pl.ds(c * NPAD + s * RPS, RPS)])

    return _sc_propagate


_sc_prop_hid = _make_sc_propagate(HID)
_sc_prop_cls = _make_sc_propagate(HID)


def _dot(a, b):
    return lax.dot_general(a, b, (((1,), (0,)), ((), ())),
                           precision=lax.Precision.HIGHEST,
                           preferred_element_type=jnp.float32)


def _dinv(d0, d1):
    return lax.rsqrt(d0 + d1 + 1.0)


def _tc_xw_body(x_ref, w_ref, o_ref):
    o_ref[...] = _dot(x_ref[...], w_ref[...])


def _tc_scale_body(xw_ref, d0_ref, d1_ref, o_ref):
    o_ref[...] = xw_ref[...] * _dinv(d0_ref[...], d1_ref[...])


def _tc_mid_body(p_ref, y_ref, d0_ref, d1_ref, b_ref, w_ref, o_ref):
    dinv = _dinv(d0_ref[...], d1_ref[...])
    t = p_ref[0:N] + p_ref[NPAD:NPAD + N] + y_ref[...]
    h = jnp.maximum(dinv * t + b_ref[...], 0.0)
    o_ref[...] = dinv * _dot(h, w_ref[...])


def _tc_final_body(p_ref, y_ref, d0_ref, d1_ref, b_ref, ls_ref, z_ref, sm_ref):
    dinv = _dinv(d0_ref[...], d1_ref[...])
    t = p_ref[0:N, 0:NCLS] + p_ref[NPAD:NPAD + N, 0:NCLS] + y_ref[0:N, 0:NCLS]
    z = dinv * t + b_ref[...]
    z_ref[...] = z
    m1 = jnp.max(z, axis=1, keepdims=True)
    e1 = jnp.exp(z - m1)
    ls_ref[...] = z - m1 - jnp.log(jnp.sum(e1, axis=1, keepdims=True))
    m0 = jnp.max(z, axis=0, keepdims=True)
    e0 = jnp.exp(z - m0)
    sm_ref[...] = e0 / jnp.sum(e0, axis=0, keepdims=True)


def _f32(shape):
    return jax.ShapeDtypeStruct(shape, jnp.float32)


_tc_xw = pl.pallas_call(_tc_xw_body, out_shape=_f32((N, HID)))
_tc_scale = pl.pallas_call(_tc_scale_body, out_shape=_f32((N, HID)))
_tc_mid = pl.pallas_call(_tc_mid_body, out_shape=_f32((N, HID)))
_tc_final = pl.pallas_call(
    _tc_final_body,
    out_shape=(_f32((N, NCLS)), _f32((N, NCLS)), _f32((N, NCLS))),
)


def kernel(x, edge_index, W1, b1, W2, b2):
    src = edge_index[0]
    dst = edge_index[1]
    dzeros = jnp.zeros((DPAD,), jnp.float32)
    ones = jnp.ones((W,), jnp.float32)
    pz_hid = jnp.zeros((NPAD, HID), jnp.float32)
    pz_cls = jnp.zeros((NPAD, HID), jnp.float32)

    pad = ((0, 0), (0, EPWP - EPW))
    src3 = jnp.pad(src.reshape(NW, EPW), pad).reshape(NW, NWIN, W)
    dst3 = jnp.pad(dst.reshape(NW, EPW), pad,
                   constant_values=NPAD - 1).reshape(NW, NWIN, W)
    dp = _sc_degree(dst3, dzeros, ones)
    xw1 = _tc_xw(x, W1)
    d0 = dp[0:N].reshape(N, 1)
    d1 = dp[DPAD:DPAD + N].reshape(N, 1)
    y1 = _tc_scale(xw1, d0, d1)
    p1 = _sc_prop_hid(y1, src3, dst3, pz_hid)
    w2p = jnp.pad(W2, ((0, 0), (0, HID - NCLS)))
    y2 = _tc_mid(p1, y1, d0, d1, b1.reshape(1, HID), w2p)
    p2 = _sc_prop_cls(y2, src3, dst3, pz_cls)
    ls, z, sm = _tc_final(p2, y2, d0, d1, b2.reshape(1, NCLS))
    return (ls, z, sm)


# speedup vs baseline: 3.2098x; 1.0333x over previous
import functools

import jax
import jax.numpy as jnp
from jax import lax
from jax.experimental import pallas as pl
from jax.experimental.pallas import tpu as pltpu
from jax.experimental.pallas import tpu_sc as plsc

N = 10000
E = 320000
F_IN = 128
HID = 128
NCLS = 64

NC = 2
NS = 16
NW = NC * NS
EPW = E // NW
W = 80
EPWP = 10000
NWIN = EPWP // W
NPAD = 10240
RPS = NPAD // NS

DPAD = 10240
DRPS = DPAD // NS


def _mesh():
    return plsc.VectorSubcoreMesh(core_axis_name="c", subcore_axis_name="s")


@functools.partial(
    pl.kernel,
    out_type=jax.ShapeDtypeStruct((NC * DPAD,), jnp.float32),
    mesh=_mesh(),
    scratch_types=[
        pltpu.VMEM((NWIN, W), jnp.int32),
        pltpu.VMEM((W,), jnp.float32),
        pltpu.VMEM_SHARED((DPAD,), jnp.float32),
    ],
)
def _sc_degree(e_hbm, z_hbm, ones_hbm, out_hbm, idx_v, ones_v, acc):
    c = lax.axis_index("c")
    s = lax.axis_index("s")
    wid = s * NC + c
    pltpu.sync_copy(ones_hbm, ones_v)
    pltpu.sync_copy(e_hbm.at[1, wid], idx_v)
    pltpu.sync_copy(z_hbm.at[pl.ds(s * DRPS, DRPS)],
                    acc.at[pl.ds(s * DRPS, DRPS)])
    plsc.subcore_barrier()

    @pl.loop(0, NWIN)
    def _(w):
        pltpu.sync_copy(ones_v, acc.at[idx_v.at[w]], add=True)

    plsc.subcore_barrier()
    pltpu.sync_copy(acc.at[pl.ds(s * DRPS, DRPS)],
                    out_hbm.at[pl.ds(c * DPAD + s * DRPS, DRPS)])


NIB = 8
NRB = 4


def _make_sc_propagate(f):
    @functools.partial(
        pl.kernel,
        out_type=jax.ShapeDtypeStruct((NC * NPAD, f), jnp.float32),
        mesh=_mesh(),
        scratch_types=[
            [pltpu.VMEM((W,), jnp.int32)] * NIB,
            [pltpu.VMEM((W,), jnp.int32)] * NIB,
            [pltpu.VMEM((W, f), jnp.float32)] * NRB,
            [pltpu.SemaphoreType.DMA] * NIB,
            [pltpu.SemaphoreType.DMA] * NRB,
            pltpu.VMEM_SHARED((NPAD, f), jnp.float32),
        ],
    )
    def _sc_propagate(y_hbm, e_hbm, z_hbm, out_hbm,
                      src_v, dst_v, rows_v, isems, rsems, acc):
        c = lax.axis_index("c")
        s = lax.axis_index("s")
        wid = s * NC + c
        pltpu.sync_copy(z_hbm.at[pl.ds(s * RPS, RPS)],
                        acc.at[pl.ds(s * RPS, RPS)])
        plsc.subcore_barrier()

        for b in range(NIB):
            pltpu.async_copy(e_hbm.at[0, wid, b], src_v[b], isems[b])
            pltpu.async_copy(e_hbm.at[1, wid, b], dst_v[b], isems[b])
        for b in range(NRB):
            pltpu.make_async_copy(e_hbm.at[0, wid, b], src_v[b],
                                  isems[b]).wait()
            pltpu.make_async_copy(e_hbm.at[1, wid, b], dst_v[b],
                                  isems[b]).wait()
            pltpu.async_copy(y_hbm.at[src_v[b]], rows_v[b], rsems[b])

        @pl.loop(0, NWIN, step=NIB)
        def _(g):
            for b in range(NIB):
                w = g + b
                rb = b % NRB
                b2 = (b + NRB) % NIB
                @pl.when(w < NWIN)
                def _():
                    pltpu.make_async_copy(
                        y_hbm.at[src_v[b]], rows_v[rb], rsems[rb]).wait()
                    pltpu.sync_copy(rows_v[rb], acc.at[dst_v[b]], add=True)

                @pl.when(w + NIB < NWIN)
                def _():
                    pltpu.async_copy(e_hbm.at[0, wid, w + NIB],
                                     src_v[b], isems[b])
                    pltpu.async_copy(e_hbm.at[1, wid, w + NIB],
                                     dst_v[b], isems[b])

                @pl.when(w + NRB < NWIN)
                def _():
                    pltpu.make_async_copy(
                        e_hbm.at[0, wid, w + NRB], src_v[b2], isems[b2]).wait()
                    pltpu.make_async_copy(
                        e_hbm.at[1, wid, w + NRB], dst_v[b2], isems[b2]).wait()
                    pltpu.async_copy(y_hbm.at[src_v[b2]], rows_v[rb],
                                     rsems[rb])

        plsc.subcore_barrier()
        pltpu.sync_copy(acc.at[pl.ds(s * RPS, RPS)],
                        out_hbm.at[pl.ds(c * NPAD + s * RPS, RPS)])

    return _sc_propagate


_sc_prop_hid = _make_sc_propagate(HID)
_sc_prop_cls = _make_sc_propagate(HID)


def _dot(a, b):
    return lax.dot_general(a, b, (((1,), (0,)), ((), ())),
                           precision=lax.Precision.HIGHEST,
                           preferred_element_type=jnp.float32)


def _dinv(d0, d1):
    return lax.rsqrt(d0 + d1 + 1.0)


def _tc_xw_body(x_ref, w_ref, o_ref):
    o_ref[...] = _dot(x_ref[...], w_ref[...])


def _tc_scale_body(xw_ref, d0_ref, d1_ref, o_ref):
    o_ref[...] = xw_ref[...] * _dinv(d0_ref[...], d1_ref[...])


def _tc_mid_body(p_ref, y_ref, d0_ref, d1_ref, b_ref, w_ref, o_ref):
    dinv = _dinv(d0_ref[...], d1_ref[...])
    t = p_ref[0:N] + p_ref[NPAD:NPAD + N] + y_ref[...]
    h = jnp.maximum(dinv * t + b_ref[...], 0.0)
    o_ref[...] = dinv * _dot(h, w_ref[...])


def _tc_final_body(p_ref, y_ref, d0_ref, d1_ref, b_ref, ls_ref, z_ref, sm_ref):
    dinv = _dinv(d0_ref[...], d1_ref[...])
    t = p_ref[0:N, 0:NCLS] + p_ref[NPAD:NPAD + N, 0:NCLS] + y_ref[0:N, 0:NCLS]
    z = dinv * t + b_ref[...]
    z_ref[...] = z
    m1 = jnp.max(z, axis=1, keepdims=True)
    e1 = jnp.exp(z - m1)
    ls_ref[...] = z - m1 - jnp.log(jnp.sum(e1, axis=1, keepdims=True))
    m0 = jnp.max(z, axis=0, keepdims=True)
    e0 = jnp.exp(z - m0)
    sm_ref[...] = e0 / jnp.sum(e0, axis=0, keepdims=True)


def _f32(shape):
    return jax.ShapeDtypeStruct(shape, jnp.float32)


_tc_xw = pl.pallas_call(_tc_xw_body, out_shape=_f32((N, HID)))
_tc_scale = pl.pallas_call(_tc_scale_body, out_shape=_f32((N, HID)))
_tc_mid = pl.pallas_call(_tc_mid_body, out_shape=_f32((N, HID)))
_tc_final = pl.pallas_call(
    _tc_final_body,
    out_shape=(_f32((N, NCLS)), _f32((N, NCLS)), _f32((N, NCLS))),
)


def kernel(x, edge_index, W1, b1, W2, b2):
    dzeros = jnp.zeros((DPAD,), jnp.float32)
    ones = jnp.ones((W,), jnp.float32)
    pz_hid = jnp.zeros((NPAD, HID), jnp.float32)
    pz_cls = jnp.zeros((NPAD, HID), jnp.float32)

    e4 = edge_index.reshape(2, NW, NWIN, W)
    dp = _sc_degree(e4, dzeros, ones)
    xw1 = _tc_xw(x, W1)
    d0 = dp[0:N].reshape(N, 1)
    d1 = dp[DPAD:DPAD + N].reshape(N, 1)
    y1 = _tc_scale(xw1, d0, d1)
    p1 = _sc_prop_hid(y1, e4, pz_hid)
    w2p = jnp.pad(W2, ((0, 0), (0, HID - NCLS)))
    y2 = _tc_mid(p1, y1, d0, d1, b1.reshape(1, HID), w2p)
    p2 = _sc_prop_cls(y2, e4, pz_cls)
    ls, z, sm = _tc_final(p2, y2, d0, d1, b2.reshape(1, NCLS))
    return (ls, z, sm)
